# baseline (speedup 1.0000x reference)
import functools

import jax
import jax.numpy as jnp
import numpy as np
from jax.experimental import pallas as pl
from jax.experimental.pallas import tpu as pltpu

_LEVELS = np.tile(np.array([8.0, 5.0, 5.0, 5.0], dtype=np.float32), 2)
_HALF_W8 = np.floor(_LEVELS / 2.0).astype(np.float32)
_EPS = 1e-3
_HALF_L8 = ((_LEVELS - 1.0) * (1.0 - _EPS) / 2.0).astype(np.float32)
_OFFSET8 = np.where(_LEVELS.astype(np.int64) % 2 == 0, 0.5, 0.0).astype(np.float32)
_SHIFT8 = np.arctanh(_OFFSET8 / _HALF_L8).astype(np.float32)

_SIZES = (3, 3, 4, 3, 3, 1, 1)


def _mm(x, w_ref, b_ref):
    return jnp.dot(x, w_ref[...], preferred_element_type=jnp.float32) + b_ref[...]


def _mlp(x, refs):
    n = len(refs) // 2
    for i in range(n):
        x = _mm(x, refs[2 * i], refs[2 * i + 1])
        if i < n - 1:
            x = jnp.maximum(x, 0.0)
    return x


def _body(x_ref, *refs):
    groups = []
    k = 0
    for s in _SIZES:
        groups.append(refs[k:k + 2 * s])
        k += 2 * s
    enc, dec, la, adec, dyn, fin, fout = groups
    shift8_ref, half_l8_ref, offset8_ref, inv_half_w8_ref = refs[k:k + 4]
    xr_ref, ap_ref, zx_ref, pz_ref = refs[k + 4:]
    shift8 = shift8_ref[...]
    half_l8 = half_l8_ref[...]
    offset8 = offset8_ref[...]
    inv_half_w8 = inv_half_w8_ref[...]

    BB, T, D = x_ref.shape
    x3 = x_ref[...]
    x2 = x3.reshape(BB * T, D)

    zx2 = _mlp(x2, enc)
    zx_ref[...] = zx2.reshape(BB, T, 16)
    xr_ref[...] = _mlp(zx2, dec).reshape(BB, T, D)

    zx3 = zx2.reshape(BB, T, 16)
    z16 = jnp.zeros((BB, 2, 16), jnp.float32)
    zx_next = jnp.concatenate([zx3[:, 2:, :], z16], axis=1)
    pair = jnp.concatenate([zx3, zx_next], axis=-1).reshape(BB * T, 32)
    za = _mlp(pair, la)

    zp = _mm(za, fin[0], fin[1])
    bounded = jnp.tanh(zp + shift8) * half_l8 - offset8
    rounded = jnp.round(bounded)
    quant = bounded + (rounded - bounded)
    vq = _mm(quant * inv_half_w8, fout[0], fout[1])

    zx_mid = jnp.concatenate([zx3[:, 1:, :], z16[:, :1, :]], axis=1).reshape(BB * T, 16)
    pz = _mlp(jnp.concatenate([vq, zx_mid], axis=-1), dyn)
    pz_ref[...] = pz.reshape(BB, T, 16)[:, : T - 2, :]

    x_mid = jnp.concatenate([x3[:, 1:, :], jnp.zeros((BB, 1, D), jnp.float32)], axis=1)
    ap = _mlp(jnp.concatenate([vq, x_mid.reshape(BB * T, D)], axis=-1), adec)
    ap_ref[...] = ap.reshape(BB, T, 64)[:, : T - 2, :]


@jax.jit
def kernel(x_ref, enc, dec, la, adec, dyn, fsq_pin, fsq_pout):
    B, T, D = x_ref.shape
    BB = 4
    grid = (B // BB,)

    flat_params = []
    for grp in (enc, dec, la, adec, dyn, [fsq_pin], [fsq_pout]):
        for (W, b) in grp:
            flat_params.append(W)
            flat_params.append(b.reshape(1, -1))

    fsq_consts = [
        jnp.asarray(_SHIFT8).reshape(1, 8),
        jnp.asarray(_HALF_L8).reshape(1, 8),
        jnp.asarray(_OFFSET8).reshape(1, 8),
        jnp.asarray(1.0 / _HALF_W8).reshape(1, 8),
    ]

    def _full(a):
        return pl.BlockSpec(a.shape, lambda i: (0,) * a.ndim)

    in_specs = [pl.BlockSpec((BB, T, D), lambda i: (i, 0, 0))]
    in_specs += [_full(a) for a in flat_params]
    in_specs += [_full(a) for a in fsq_consts]

    out_shapes = (
        jax.ShapeDtypeStruct((B, T, D), jnp.float32),
        jax.ShapeDtypeStruct((B, T - 2, 64), jnp.float32),
        jax.ShapeDtypeStruct((B, T, 16), jnp.float32),
        jax.ShapeDtypeStruct((B, T - 2, 16), jnp.float32),
    )
    out_specs = (
        pl.BlockSpec((BB, T, D), lambda i: (i, 0, 0)),
        pl.BlockSpec((BB, T - 2, 64), lambda i: (i, 0, 0)),
        pl.BlockSpec((BB, T, 16), lambda i: (i, 0, 0)),
        pl.BlockSpec((BB, T - 2, 16), lambda i: (i, 0, 0)),
    )

    return pl.pallas_call(
        _body,
        grid=grid,
        in_specs=in_specs,
        out_specs=out_specs,
        out_shape=out_shapes,
        compiler_params=pltpu.CompilerParams(
            dimension_semantics=("arbitrary",),
        ),
    )(x_ref, *flat_params, *fsq_consts)


# speedup vs baseline: 1.0678x; 1.0678x over previous
import functools

import jax
import jax.numpy as jnp
import numpy as np
from jax.experimental import pallas as pl
from jax.experimental.pallas import tpu as pltpu

_LEVELS = np.tile(np.array([8.0, 5.0, 5.0, 5.0], dtype=np.float32), 2)
_HALF_W8 = np.floor(_LEVELS / 2.0).astype(np.float32)
_EPS = 1e-3
_HALF_L8 = ((_LEVELS - 1.0) * (1.0 - _EPS) / 2.0).astype(np.float32)
_OFFSET8 = np.where(_LEVELS.astype(np.int64) % 2 == 0, 0.5, 0.0).astype(np.float32)
_SHIFT8 = np.arctanh(_OFFSET8 / _HALF_L8).astype(np.float32)

_SIZES = (3, 3, 4, 3, 3, 1, 1)


def _mm(x, w_ref, b_ref):
    return jnp.dot(x, w_ref[...], preferred_element_type=jnp.float32) + b_ref[...]


def _mlp(x, refs):
    n = len(refs) // 2
    for i in range(n):
        x = _mm(x, refs[2 * i], refs[2 * i + 1])
        if i < n - 1:
            x = jnp.maximum(x, 0.0)
    return x


def _body(x_ref, *refs):
    groups = []
    k = 0
    for s in _SIZES:
        groups.append(refs[k:k + 2 * s])
        k += 2 * s
    enc, dec, la, adec, dyn, fin, fout = groups
    shift8_ref, half_l8_ref, offset8_ref, inv_half_w8_ref = refs[k:k + 4]
    xr_ref, ap_ref, zx_ref, pz_ref = refs[k + 4:]
    shift8 = shift8_ref[...]
    half_l8 = half_l8_ref[...]
    offset8 = offset8_ref[...]
    inv_half_w8 = inv_half_w8_ref[...]

    BB, T, D = x_ref.shape
    x3 = x_ref[...]
    x2 = x3.reshape(BB * T, D)

    zx2 = _mlp(x2, enc)
    zx_ref[...] = zx2.reshape(BB, T, 16)
    xr_ref[...] = _mlp(zx2, dec).reshape(BB, T, D)

    zx3 = zx2.reshape(BB, T, 16)
    z16 = jnp.zeros((BB, 2, 16), jnp.float32)
    zx_next = jnp.concatenate([zx3[:, 2:, :], z16], axis=1)
    pair = jnp.concatenate([zx3, zx_next], axis=-1).reshape(BB * T, 32)
    za = _mlp(pair, la)

    zp = _mm(za, fin[0], fin[1])
    bounded = jnp.tanh(zp + shift8) * half_l8 - offset8
    rounded = jnp.round(bounded)
    quant = bounded + (rounded - bounded)
    vq = _mm(quant * inv_half_w8, fout[0], fout[1])

    zx_mid = jnp.concatenate([zx3[:, 1:, :], z16[:, :1, :]], axis=1).reshape(BB * T, 16)
    pz = _mlp(jnp.concatenate([vq, zx_mid], axis=-1), dyn)
    pz_ref[...] = pz.reshape(BB, T, 16)[:, : T - 2, :]

    x_mid = jnp.concatenate([x3[:, 1:, :], jnp.zeros((BB, 1, D), jnp.float32)], axis=1)
    ap = _mlp(jnp.concatenate([vq, x_mid.reshape(BB * T, D)], axis=-1), adec)
    ap_ref[...] = ap.reshape(BB, T, 64)[:, : T - 2, :]


@jax.jit
def kernel(x_ref, enc, dec, la, adec, dyn, fsq_pin, fsq_pout):
    B, T, D = x_ref.shape
    BB = 16
    grid = (B // BB,)

    flat_params = []
    for grp in (enc, dec, la, adec, dyn, [fsq_pin], [fsq_pout]):
        for (W, b) in grp:
            flat_params.append(W)
            flat_params.append(b.reshape(1, -1))

    fsq_consts = [
        jnp.asarray(_SHIFT8).reshape(1, 8),
        jnp.asarray(_HALF_L8).reshape(1, 8),
        jnp.asarray(_OFFSET8).reshape(1, 8),
        jnp.asarray(1.0 / _HALF_W8).reshape(1, 8),
    ]

    def _full(a):
        return pl.BlockSpec(a.shape, lambda i: (0,) * a.ndim)

    in_specs = [pl.BlockSpec((BB, T, D), lambda i: (i, 0, 0))]
    in_specs += [_full(a) for a in flat_params]
    in_specs += [_full(a) for a in fsq_consts]

    out_shapes = (
        jax.ShapeDtypeStruct((B, T, D), jnp.float32),
        jax.ShapeDtypeStruct((B, T - 2, 64), jnp.float32),
        jax.ShapeDtypeStruct((B, T, 16), jnp.float32),
        jax.ShapeDtypeStruct((B, T - 2, 16), jnp.float32),
    )
    out_specs = (
        pl.BlockSpec((BB, T, D), lambda i: (i, 0, 0)),
        pl.BlockSpec((BB, T - 2, 64), lambda i: (i, 0, 0)),
        pl.BlockSpec((BB, T, 16), lambda i: (i, 0, 0)),
        pl.BlockSpec((BB, T - 2, 16), lambda i: (i, 0, 0)),
    )

    return pl.pallas_call(
        _body,
        grid=grid,
        in_specs=in_specs,
        out_specs=out_specs,
        out_shape=out_shapes,
        compiler_params=pltpu.CompilerParams(
            dimension_semantics=("arbitrary",),
        ),
    )(x_ref, *flat_params, *fsq_consts)
